# baseline (speedup 1.0000x reference)
import functools

import jax
import jax.numpy as jnp
from jax import lax
from jax.experimental import pallas as pl
from jax.experimental.pallas import tpu as pltpu
from jax.experimental.pallas import tpu_sc as plsc

B = 16384
F = 26
D = 32

NC, NS = 2, 16
NW = NC * NS
BT_PER_W = B // 128 // NW
FB = 6
FBLOCKS = ((0, 6), (6, 6), (12, 6), (18, 6), (24, 2))
OP = 129

_mesh = plsc.VectorSubcoreMesh(core_axis_name="c", subcore_axis_name="s")

TCG = 1024
TCN = (1000000 + TCG - 1) // TCG
TROWS = TCN * TCG


def _linearize_body(t_ref, o_ref):
    o_ref[:, :D] = t_ref[...]


_linearize = pl.pallas_call(
    _linearize_body,
    out_shape=jax.ShapeDtypeStruct((TROWS, 128), jnp.float32),
    grid=(TCN,),
    in_specs=[pl.BlockSpec((TCG, D), lambda j: (j, 0))],
    out_specs=pl.BlockSpec((TCG, 128), lambda j: (j, 0)),
)


@functools.partial(
    pl.kernel,
    out_type=jax.ShapeDtypeStruct((F, D // 8, B // 128, 8, 128), jnp.float32),
    mesh=_mesh,
    compiler_params=pltpu.CompilerParams(
        use_tc_tiling_on_sc=False, needs_layout_passes=False),
    scratch_types=[
        pltpu.VMEM((2, 4, 8, 128), jnp.int32),
        pltpu.VMEM((2, FB * 128, D), jnp.float32),
        pltpu.VMEM((2, FB, D // 8, 8, OP), jnp.float32),
        pltpu.VMEM((F * D,), jnp.float32),
        pltpu.SemaphoreType.DMA,
        pltpu.SemaphoreType.DMA,
        pltpu.SemaphoreType.DMA,
        pltpu.SemaphoreType.DMA,
        pltpu.SemaphoreType.DMA,
    ],
)
def _embed(x4_hbm, table_hbm, bias_hbm, out_hbm, idx_v, rows_v, o_v, bias_v,
           isem, gsem0, gsem1, osem0, osem1):
    gsems = (gsem0, gsem1)
    osems = (osem0, osem1)
    wid = lax.axis_index("s") * NC + lax.axis_index("c")

    pltpu.sync_copy(bias_hbm, bias_v)

    blocks = [(j, f0, fb) for j in range(BT_PER_W) for (f0, fb) in FBLOCKS]
    nblk = len(blocks)

    def load_x(j):
        return pltpu.async_copy(
            x4_hbm.at[:, wid * BT_PER_W + j], idx_v.at[j % 2], isem)

    def start_block(s, idescs):
        j, f0, fb = blocks[s]
        p = s % 2
        if s % len(FBLOCKS) == 0:
            idescs[j % 2].wait()
            if j + 1 < BT_PER_W:
                idescs[(j + 1) % 2] = load_x(j + 1)
        descs = []
        for fi in range(fb):
            f = f0 + fi
            descs.append(pltpu.async_copy(
                table_hbm.at[idx_v.at[j % 2, f // 8, f % 8]],
                rows_v.at[p, pl.ds(fi * 128, 128)],
                gsems[p]))
        return descs

    viota = lax.iota(jnp.int32, 16)
    dt0 = viota // 8
    dr0 = viota % 8
    d1 = viota + 16
    dt1 = d1 // 8
    dr1 = d1 % 8

    def compute_block(s):
        _, f0, fb = blocks[s]
        p = s % 2

        def fi_body(fi, carry):
            f = f0 + fi
            b0 = bias_v[pl.ds(f * D, 16)]
            b1 = bias_v[pl.ds(f * D + 16, 16)]
            fiv = jnp.full((16,), fi, jnp.int32)

            def u_body(u, carry2):
                for k in range(4):
                    bb = u * 4 + k
                    r = fi * 128 + bb
                    bbv = jnp.full((16,), bb, jnp.int32)
                    v0 = rows_v[p, r, pl.ds(0, 16)] + b0
                    plsc.store_scatter(o_v.at[p], [fiv, dt0, dr0, bbv], v0)
                    v1 = rows_v[p, r, pl.ds(16, 16)] + b1
                    plsc.store_scatter(o_v.at[p], [fiv, dt1, dr1, bbv], v1)
                return carry2
            lax.fori_loop(0, 32, u_body, 0)
            return carry
        lax.fori_loop(0, fb, fi_body, 0)

    idescs = [None, None]
    gdescs = [None, None]
    odescs = [None, None]
    idescs[0] = load_x(0)
    gdescs[0] = start_block(0, idescs)
    for s in range(nblk):
        p = s % 2
        if s + 1 < nblk:
            gdescs[1 - p] = start_block(s + 1, idescs)
        for dsc in gdescs[p]:
            dsc.wait()
        if odescs[p] is not None:
            odescs[p].wait()
        compute_block(s)
        j, f0, fb = blocks[s]
        bt = wid * BT_PER_W + j
        odescs[p] = pltpu.async_copy(
            o_v.at[p, pl.ds(0, fb), :, :, pl.ds(0, 128)],
            out_hbm.at[pl.ds(f0, fb), :, bt, :, :],
            osems[p])
    odescs[0].wait()
    odescs[1].wait()


def kernel(x, table, bias):
    xp = jnp.pad(x.astype(jnp.int32) * 4, ((0, 0), (0, 6)))
    x4 = xp.T.reshape(4, 8, 128, 128).transpose(0, 2, 1, 3)
    tv = _linearize(table).reshape(4 * TROWS, D)
    out = _embed(x4, tv, bias.reshape(F * D))
    return out.transpose(2, 4, 0, 1, 3).reshape(B, F, D)


# speedup vs baseline: 1.8136x; 1.8136x over previous
import functools

import jax
import jax.numpy as jnp
from jax import lax
from jax.experimental import pallas as pl
from jax.experimental.pallas import tpu as pltpu
from jax.experimental.pallas import tpu_sc as plsc

B = 16384
F = 26
D = 32
V = 1000000
VP = 1000064
CT = VP // 128

NC, NS = 2, 16
NW = NC * NS
BT_PER_W = B // 128 // NW
FB = 6
FBLOCKS = ((0, 6), (6, 6), (12, 6), (18, 6), (24, 2))
OP = 129

RG = 6
RNB = 42
RPER = (CT + NW - 1) // NW
RP = 129

_mesh = plsc.VectorSubcoreMesh(core_axis_name="c", subcore_axis_name="s")
_params = pltpu.CompilerParams(
    use_tc_tiling_on_sc=False, needs_layout_passes=False)


@functools.partial(
    pl.kernel,
    out_type=jax.ShapeDtypeStruct((VP, D), jnp.float32),
    mesh=_mesh,
    compiler_params=_params,
    scratch_types=[
        pltpu.VMEM((2, D // 8, RG, 8, RP), jnp.float32),
        pltpu.VMEM((2, RG * 128, D), jnp.float32),
        pltpu.SemaphoreType.DMA,
        pltpu.SemaphoreType.DMA,
        pltpu.SemaphoreType.DMA,
        pltpu.SemaphoreType.DMA,
    ],
)
def _relayout(tp4_hbm, out_hbm, inb, stage, isem0, isem1, osem0, osem1):
    isems = (isem0, isem1)
    osems = (osem0, osem1)
    wid = lax.axis_index("s") * NC + lax.axis_index("c")
    viota = lax.iota(jnp.int32, 16)

    def it0_of(bi):
        return jnp.minimum(wid * RPER + bi * RG, CT - RG)

    def in_copy(bi, p):
        return pltpu.make_async_copy(
            tp4_hbm.at[:, pl.ds(it0_of(bi), RG)],
            inb.at[p, :, :, :, pl.ds(0, 128)], isems[p])

    def out_copy(bi, p):
        return pltpu.make_async_copy(
            stage.at[p],
            out_hbm.at[pl.ds(it0_of(bi) * 128, RG * 128)],
            osems[p])

    dtv0 = viota // 8
    drv0 = viota % 8
    dtv1 = (viota + 16) // 8
    drv1 = (viota + 16) % 8

    def compute(p):
        def g_body(g, carry):
            gv = jnp.full((16,), g, jnp.int32)

            def u_body(u, carry2):
                for k in range(4):
                    ic = u * 4 + k
                    icv = jnp.full((16,), ic, jnp.int32)
                    r = g * 128 + ic
                    v0 = plsc.load_gather(inb.at[p], [dtv0, gv, drv0, icv])
                    stage[p, r, pl.ds(0, 16)] = v0
                    v1 = plsc.load_gather(inb.at[p], [dtv1, gv, drv1, icv])
                    stage[p, r, pl.ds(16, 16)] = v1
                return carry2
            lax.fori_loop(0, 32, u_body, 0)
            return carry
        lax.fori_loop(0, RG, g_body, 0)

    in_copy(0, 0).start()
    in_copy(1, 1).start()

    def b_body(b, carry):
        for p in range(2):
            bi = b * 2 + p
            in_copy(bi, p).wait()
            @pl.when(bi >= 2)
            def _drain():
                out_copy(bi - 2, p).wait()
            compute(p)
            out_copy(bi, p).start()
            @pl.when(bi + 2 < RNB)
            def _next():
                in_copy(bi + 2, p).start()
        return carry
    lax.fori_loop(0, RNB // 2, b_body, 0)
    out_copy(RNB - 2, 0).wait()
    out_copy(RNB - 1, 1).wait()


@functools.partial(
    pl.kernel,
    out_type=jax.ShapeDtypeStruct((F, D // 8, B // 128, 8, 128), jnp.float32),
    mesh=_mesh,
    compiler_params=_params,
    scratch_types=[
        pltpu.VMEM((2, 4, 8, 128), jnp.int32),
        pltpu.VMEM((2, FB * 128, D), jnp.float32),
        pltpu.VMEM((2, FB, D // 8, 8, OP), jnp.float32),
        pltpu.VMEM((F * D,), jnp.float32),
        pltpu.SemaphoreType.DMA,
        pltpu.SemaphoreType.DMA,
        pltpu.SemaphoreType.DMA,
        pltpu.SemaphoreType.DMA,
        pltpu.SemaphoreType.DMA,
    ],
)
def _embed(x4_hbm, table_hbm, bias_hbm, out_hbm, idx_v, rows_v, o_v, bias_v,
           isem, gsem0, gsem1, osem0, osem1):
    gsems = (gsem0, gsem1)
    osems = (osem0, osem1)
    wid = lax.axis_index("s") * NC + lax.axis_index("c")

    pltpu.sync_copy(bias_hbm, bias_v)

    blocks = [(j, f0, fb) for j in range(BT_PER_W) for (f0, fb) in FBLOCKS]
    nblk = len(blocks)

    def load_x(j):
        return pltpu.async_copy(
            x4_hbm.at[:, wid * BT_PER_W + j], idx_v.at[j % 2], isem)

    def start_block(s, idescs):
        j, f0, fb = blocks[s]
        p = s % 2
        if s % len(FBLOCKS) == 0:
            idescs[j % 2].wait()
            if j + 1 < BT_PER_W:
                idescs[(j + 1) % 2] = load_x(j + 1)
        descs = []
        for fi in range(fb):
            f = f0 + fi
            descs.append(pltpu.async_copy(
                table_hbm.at[idx_v.at[j % 2, f // 8, f % 8]],
                rows_v.at[p, pl.ds(fi * 128, 128)],
                gsems[p]))
        return descs

    viota = lax.iota(jnp.int32, 16)
    dt0 = viota // 8
    dr0 = viota % 8
    d1 = viota + 16
    dt1 = d1 // 8
    dr1 = d1 % 8

    def compute_block(s):
        _, f0, fb = blocks[s]
        p = s % 2

        def fi_body(fi, carry):
            f = f0 + fi
            b0 = bias_v[pl.ds(f * D, 16)]
            b1 = bias_v[pl.ds(f * D + 16, 16)]
            fiv = jnp.full((16,), fi, jnp.int32)

            def u_body(u, carry2):
                for k in range(4):
                    bb = u * 4 + k
                    r = fi * 128 + bb
                    bbv = jnp.full((16,), bb, jnp.int32)
                    v0 = rows_v[p, r, pl.ds(0, 16)] + b0
                    plsc.store_scatter(o_v.at[p], [fiv, dt0, dr0, bbv], v0)
                    v1 = rows_v[p, r, pl.ds(16, 16)] + b1
                    plsc.store_scatter(o_v.at[p], [fiv, dt1, dr1, bbv], v1)
                return carry2
            lax.fori_loop(0, 32, u_body, 0)
            return carry
        lax.fori_loop(0, fb, fi_body, 0)

    idescs = [None, None]
    gdescs = [None, None]
    odescs = [None, None]
    idescs[0] = load_x(0)
    gdescs[0] = start_block(0, idescs)
    for s in range(nblk):
        p = s % 2
        if s + 1 < nblk:
            gdescs[1 - p] = start_block(s + 1, idescs)
        for dsc in gdescs[p]:
            dsc.wait()
        if odescs[p] is not None:
            odescs[p].wait()
        compute_block(s)
        j, f0, fb = blocks[s]
        bt = wid * BT_PER_W + j
        odescs[p] = pltpu.async_copy(
            o_v.at[p, pl.ds(0, fb), :, :, pl.ds(0, 128)],
            out_hbm.at[pl.ds(f0, fb), :, bt, :, :],
            osems[p])
    odescs[0].wait()
    odescs[1].wait()


def kernel(x, table, bias):
    xp = jnp.pad(x.astype(jnp.int32), ((0, 0), (0, 6)))
    x4 = xp.T.reshape(4, 8, 128, 128).transpose(0, 2, 1, 3)
    tpad = jnp.pad(table, ((0, VP - V), (0, 0)))
    tp4 = tpad.T.reshape(4, 8, CT, 128).transpose(0, 2, 1, 3)
    tlin = _relayout(tp4)
    out = _embed(x4, tlin, bias.reshape(F * D))
    return out.transpose(2, 4, 0, 1, 3).reshape(B, F, D)
